# Initial kernel scaffold; baseline (speedup 1.0000x reference)
import jax
import jax.numpy as jnp
from jax.experimental import pallas as pl


def kernel(xyz, points, W0, b0, g0, beta0, W1, b1, g1, beta1, W2, b2, g2, beta2):
    raise NotImplementedError("write your pallas kernel here")


# speedup vs baseline: 1.0409x; 1.0409x over previous
import jax
import jax.numpy as jnp
from jax import lax
from jax.experimental import pallas as pl
from jax.experimental.pallas import tpu as pltpu

B = 8
N = 2048
TILE = 512
TPB = N // TILE
NT = B * TPB
M = B * N
EPS = 1e-5
C1, C2, C3 = 256, 512, 1024


def _body(xyz_ref, pts_ref, w0a_ref, w0b_ref, w1_ref, w2_ref,
          b0_ref, b1_ref, b2_ref,
          g0_ref, be0_ref, g1_ref, be1_ref, g2_ref, be2_ref,
          out_ref,
          y0s, y1s, s0m, s0q, s1m, s1q, s2m, s2q, ymax, ymin):
    i = pl.program_id(0)
    t = i % NT
    b = t // TPB
    inv_m = 1.0 / M

    @pl.when(i < NT)
    def _phase0():
        xv = xyz_ref[t]
        pv = pts_ref[0]
        y = jnp.dot(w0b_ref[...], pv, preferred_element_type=jnp.float32)
        y = y + jnp.dot(w0a_ref[...], xv, preferred_element_type=jnp.float32)
        y = y + b0_ref[...]
        yb = y.astype(jnp.bfloat16)
        y0s[t] = yb
        yf = yb.astype(jnp.float32)
        ps = jnp.sum(yf, axis=1, keepdims=True)
        pq = jnp.sum(yf * yf, axis=1, keepdims=True)

        @pl.when(t == 0)
        def _():
            s0m[...] = ps
            s0q[...] = pq

        @pl.when(t != 0)
        def _():
            s0m[...] += ps
            s0q[...] += pq

    @pl.when(jnp.logical_and(i >= NT, i < 2 * NT))
    def _phase1():
        mean = s0m[...] * inv_m
        var = jnp.maximum(s0q[...] * inv_m - mean * mean, 0.0)
        sc = g0_ref[...] * lax.rsqrt(var + EPS)
        sh = be0_ref[...] - mean * sc
        y0 = y0s[t].astype(jnp.float32)
        z = jnp.maximum(y0 * sc + sh, 0.0).astype(jnp.bfloat16)
        y = jnp.dot(w1_ref[...], z, preferred_element_type=jnp.float32)
        y = y + b1_ref[...]
        yb = y.astype(jnp.bfloat16)
        y1s[t] = yb
        yf = yb.astype(jnp.float32)
        ps = jnp.sum(yf, axis=1, keepdims=True)
        pq = jnp.sum(yf * yf, axis=1, keepdims=True)

        @pl.when(t == 0)
        def _():
            s1m[...] = ps
            s1q[...] = pq

        @pl.when(t != 0)
        def _():
            s1m[...] += ps
            s1q[...] += pq

    @pl.when(jnp.logical_and(i >= 2 * NT, i < 3 * NT))
    def _phase2():
        mean = s1m[...] * inv_m
        var = jnp.maximum(s1q[...] * inv_m - mean * mean, 0.0)
        sc = g1_ref[...] * lax.rsqrt(var + EPS)
        sh = be1_ref[...] - mean * sc
        y1 = y1s[t].astype(jnp.float32)
        z = jnp.maximum(y1 * sc + sh, 0.0).astype(jnp.bfloat16)
        y = jnp.dot(w2_ref[...], z, preferred_element_type=jnp.float32)
        y = y + b2_ref[...]
        ps = jnp.sum(y, axis=1, keepdims=True)
        pq = jnp.sum(y * y, axis=1, keepdims=True)
        mx = jnp.max(y, axis=1, keepdims=True)
        mn = jnp.min(y, axis=1, keepdims=True)
        lanes = lax.broadcasted_iota(jnp.int32, (C3, B), 1)
        mxb = jnp.where(lanes == b, mx, -jnp.inf)
        mnb = jnp.where(lanes == b, mn, jnp.inf)

        @pl.when(t == 0)
        def _():
            s2m[...] = ps
            s2q[...] = pq
            ymax[...] = mxb
            ymin[...] = mnb

        @pl.when(t != 0)
        def _():
            s2m[...] += ps
            s2q[...] += pq
            ymax[...] = jnp.maximum(ymax[...], mxb)
            ymin[...] = jnp.minimum(ymin[...], mnb)

    @pl.when(i == 3 * NT)
    def _finish():
        mean = s2m[...] * inv_m
        var = jnp.maximum(s2q[...] * inv_m - mean * mean, 0.0)
        sc = g2_ref[...] * lax.rsqrt(var + EPS)
        sh = be2_ref[...] - mean * sc
        ext = jnp.where(sc >= 0.0, ymax[...], ymin[...])
        out_ref[...] = jnp.maximum(ext * sc + sh, 0.0)


def kernel(xyz, points, W0, b0, g0, beta0, W1, b1, g1, beta1, W2, b2, g2, beta2):
    bf = jnp.bfloat16
    f32 = jnp.float32
    xyz_t = xyz.transpose(1, 0, 2).reshape(3, NT, TILE).transpose(1, 0, 2).astype(bf)
    pts = points.astype(bf)
    w0a = W0[:, :3].astype(bf)
    w0b = W0[:, 3:].astype(bf)
    w1 = W1.astype(bf)
    w2 = W2.astype(bf)

    def col(v):
        return v.reshape(-1, 1).astype(f32)

    grid = 3 * NT + 1
    full = lambda shape: pl.BlockSpec(shape, lambda i: tuple(0 for _ in shape))
    out = pl.pallas_call(
        _body,
        grid=(grid,),
        in_specs=[
            full((NT, 3, TILE)),
            pl.BlockSpec((1, C1, TILE),
                         lambda i: (jnp.minimum(i, NT - 1) // TPB, 0,
                                    jnp.minimum(i, NT - 1) % TPB)),
            full((C1, 3)),
            full((C1, C1)),
            full((C2, C1)),
            full((C3, C2)),
            full((C1, 1)),
            full((C2, 1)),
            full((C3, 1)),
            full((C1, 1)),
            full((C1, 1)),
            full((C2, 1)),
            full((C2, 1)),
            full((C3, 1)),
            full((C3, 1)),
        ],
        out_specs=pl.BlockSpec((C3, B), lambda i: (0, 0)),
        out_shape=jax.ShapeDtypeStruct((C3, B), f32),
        scratch_shapes=[
            pltpu.VMEM((NT, C1, TILE), bf),
            pltpu.VMEM((NT, C2, TILE), bf),
            pltpu.VMEM((C1, 1), f32),
            pltpu.VMEM((C1, 1), f32),
            pltpu.VMEM((C2, 1), f32),
            pltpu.VMEM((C2, 1), f32),
            pltpu.VMEM((C3, 1), f32),
            pltpu.VMEM((C3, 1), f32),
            pltpu.VMEM((C3, B), f32),
            pltpu.VMEM((C3, B), f32),
        ],
    )(xyz_t, pts, w0a, w0b, w1, w2,
      col(b0), col(b1), col(b2),
      col(g0), col(beta0), col(g1), col(beta1), col(g2), col(beta2))

    new_points = out.T.reshape(B, C3, 1)
    new_xyz = jnp.zeros((B, 3, 1), f32)
    return new_xyz, new_points
